# baseline (speedup 1.0000x reference)
import functools

import jax
import jax.numpy as jnp
from jax import lax
from jax.experimental import pallas as pl
from jax.experimental.pallas import tpu as pltpu
from jax.experimental.pallas import tpu_sc as plsc

_NUM_EMBEDDINGS = 1000000
_TP_SIZE = 8
_TP_RANK = 1
_PER_PART = _NUM_EMBEDDINGS // _TP_SIZE
_VOCAB_START = _PER_PART * _TP_RANK
_VOCAB_END = _VOCAB_START + _PER_PART
_EMBED_DIM = 64

_NW = 32
_L = 16
_BS = 15625
_NBLK = _PER_PART // _BS
_NSTG = 5
_SROWS = _BS // _NSTG
_FR = 64
_NSLOT = 4
_ZWIN = 16
_QROWS = 100


def _emb_call(B):
  bpw = B // _NW
  ngrp = bpw // _L
  zgrp = ngrp // _NBLK
  zcap_rows = zgrp * _L // _FR
  q = _QROWS * _FR
  mesh = plsc.VectorSubcoreMesh(core_axis_name="c", subcore_axis_name="s")

  scratch = (
      [pltpu.VMEM((bpw,), jnp.int32)]
      + [pltpu.VMEM((_QROWS, _FR), jnp.int32)]
      + [pltpu.VMEM((_QROWS, _FR), jnp.int32)]
      + [pltpu.VMEM((zcap_rows, _FR), jnp.int32)]
      + [pltpu.VMEM((_FR, _EMBED_DIM), jnp.float32) for _ in range(_NSLOT)]
      + [pltpu.VMEM((_FR, _EMBED_DIM), jnp.float32)]
      + [pltpu.SemaphoreType.DMA for _ in range(2 * _NSLOT + 2)]
      + [pltpu.VMEM_SHARED((_BS, _EMBED_DIM), jnp.float32)]
  )

  @functools.partial(
      pl.kernel,
      out_type=jax.ShapeDtypeStruct((B, _EMBED_DIM), jnp.float32),
      mesh=mesh,
      scratch_types=scratch,
      compiler_params=pltpu.CompilerParams(
          needs_layout_passes=False, use_tc_tiling_on_sc=False),
  )
  def emb(x_hbm, w_hbm, out_hbm, idx_v, gbuf, pbuf, zpbuf, *bufs):
    rows = bufs[:_NSLOT]
    zrows = bufs[_NSLOT]
    gsem = bufs[_NSLOT + 1:2 * _NSLOT + 1]
    wsem = bufs[2 * _NSLOT + 1:3 * _NSLOT + 1]
    zsem = bufs[3 * _NSLOT + 1]
    ssem = bufs[3 * _NSLOT + 2]
    w_sp = bufs[3 * _NSLOT + 3]

    sid = lax.axis_index("s")
    cid = lax.axis_index("c")
    wid = sid * 2 + cid
    base = wid * bpw
    iota = lax.iota(jnp.int32, _L)
    pltpu.sync_copy(x_hbm.at[pl.ds(base, bpw)], idx_v)

    zvec = jnp.zeros((_L,), jnp.float32)

    def zfill(i, c):
      r = jnp.full((_L,), i >> 2, jnp.int32)
      cc = (i & 3) << 4
      plsc.store_scatter(zrows, [r, cc + iota], zvec)
      return c

    lax.fori_loop(0, _FR * _EMBED_DIM // _L, zfill, 0)

    def compact(buf2, vals_fn, mask_fn, glo, ghi, wlo, cap, pos_to=None):
      def grp(i, nvv):
        v = idx_v[pl.ds(i * _L, _L)]
        m = mask_fn(v)
        cs = plsc.cumsum(m.astype(jnp.int32))
        pc = plsc.all_reduce_population_count(m)
        dst = nvv + cs - 1
        sm = m & (dst >= wlo) & (dst < wlo + cap)
        d2 = dst - wlo
        plsc.store_scatter(buf2, [d2 >> 6, d2 & 63], vals_fn(v, i), mask=sm)
        if pos_to is not None:
          posv = base + i * _L + iota
          plsc.store_scatter(pos_to, [d2 >> 6, d2 & 63], posv, mask=sm)
        return nvv + pc

      nvv = lax.fori_loop(glo, ghi, grp, jnp.zeros((_L,), jnp.int32),
                          unroll=2)
      return jnp.max(nvv)

    def tail_fill(nv, bufs2):
      last = jnp.maximum(nv - 1, 0)
      lr = jnp.full((_L,), last >> 6, jnp.int32)
      lc = jnp.full((_L,), last & 63, jnp.int32)
      r64 = ((nv + 63) >> 6) << 6
      w0 = nv - (nv & 15)
      for buf2 in bufs2:
        dup = plsc.load_gather(buf2, [lr, lc])
        for t in range(4):
          slot = w0 + t * _L + iota
          mk = (slot >= nv) & (slot < r64)
          plsc.store_scatter(buf2, [slot >> 6, slot & 63], dup, mask=mk)
      return r64 >> 6

    def fire_rounds(nf):
      def fire4(i, c):
        for j in range(_NSLOT):
          k = i * _NSLOT + j

          @pl.when(k < nf)
          def _fire():
            @pl.when(k >= _NSLOT)
            def _wait_prev():
              pltpu.make_async_copy(
                  rows[j], out_hbm.at[pbuf.at[0]], wsem[j]).wait()

            pltpu.async_copy(w_sp.at[gbuf.at[k]], rows[j], gsem[j])
            pltpu.make_async_copy(
                w_sp.at[gbuf.at[k]], rows[j], gsem[j]).wait()
            pltpu.async_copy(rows[j], out_hbm.at[pbuf.at[k]], wsem[j])

        return c

      lax.fori_loop(0, (nf + _NSLOT - 1) // _NSLOT, fire4, 0)

      for j in range(_NSLOT):
        @pl.when(nf > j)
        def _drain():
          pltpu.make_async_copy(
              rows[j], out_hbm.at[pbuf.at[0]], wsem[j]).wait()

    def zwait_one(k, c):
      pltpu.make_async_copy(zrows, out_hbm.at[zpbuf.at[0]], zsem).wait()
      return c

    zpend = jnp.int32(0)
    for p in range(_NBLK):
      lo = _VOCAB_START + p * _BS
      plsc.subcore_barrier()

      @pl.when(sid < _NSTG)
      def _stage():
        pltpu.async_copy(
            w_hbm.at[pl.ds(p * _BS + sid * _SROWS, _SROWS)],
            w_sp.at[pl.ds(sid * _SROWS, _SROWS)], ssem).wait()

      plsc.subcore_barrier()

      in_blk = lambda v: (v >= lo) & (v < lo + _BS)
      to_local = lambda v, i: v - lo
      nv_tot = compact(gbuf, to_local, in_blk, 0, ngrp, jnp.int32(0), q,
                       pos_to=pbuf)
      nv0 = jnp.minimum(nv_tot, q)
      fire_rounds(tail_fill(nv0, [gbuf, pbuf]))

      def extra_round(r, c):
        wlo = r * q
        compact(gbuf, to_local, in_blk, 0, ngrp, wlo, q, pos_to=pbuf)
        nv_r = jnp.minimum(nv_tot - wlo, q)
        fire_rounds(tail_fill(nv_r, [gbuf, pbuf]))
        return c

      lax.fori_loop(1, (nv_tot + q - 1) // q, extra_round, 0)

      lax.fori_loop(0, zpend, zwait_one, 0)
      nz = compact(
          zpbuf,
          lambda v, i: base + i * _L + iota,
          lambda v: (v < _VOCAB_START) | (v >= _VOCAB_END),
          p * zgrp, (p + 1) * zgrp, jnp.int32(0), zgrp * _L)
      nzf = tail_fill(nz, [zpbuf])

      def zfire(k, c):
        pltpu.async_copy(zrows, out_hbm.at[zpbuf.at[k]], zsem)

        @pl.when(k >= _ZWIN)
        def _zw():
          zwait_one(k, 0)

        return c

      lax.fori_loop(0, nzf, zfire, 0)
      zpend = jnp.minimum(nzf, _ZWIN)

    lax.fori_loop(0, zpend, zwait_one, 0)

  return emb


def kernel(x, weight):
  s0, s1 = x.shape
  B = s0 * s1
  xf = x.reshape(B).astype(jnp.int32)
  out = _emb_call(B)(xf, weight)
  return out.reshape(s0, s1, _EMBED_DIM)


# speedup vs baseline: 1.0172x; 1.0172x over previous
import functools

import jax
import jax.numpy as jnp
from jax import lax
from jax.experimental import pallas as pl
from jax.experimental.pallas import tpu as pltpu
from jax.experimental.pallas import tpu_sc as plsc

_NUM_EMBEDDINGS = 1000000
_TP_SIZE = 8
_TP_RANK = 1
_PER_PART = _NUM_EMBEDDINGS // _TP_SIZE
_VOCAB_START = _PER_PART * _TP_RANK
_VOCAB_END = _VOCAB_START + _PER_PART
_EMBED_DIM = 64

_NW = 32
_L = 16
_BS = 15625
_NBLK = _PER_PART // _BS
_NSTG = 5
_SROWS = _BS // _NSTG
_FR = 64
_NSLOT = 4
_ZWIN = 24
_QROWS = 100


def _emb_call(B):
  bpw = B // _NW
  ngrp = bpw // _L
  zgrp = ngrp // _NBLK
  zcap_rows = zgrp * _L // _FR
  q = _QROWS * _FR
  mesh = plsc.VectorSubcoreMesh(core_axis_name="c", subcore_axis_name="s")

  scratch = (
      [pltpu.VMEM((bpw,), jnp.int32)]
      + [pltpu.VMEM((_QROWS, _FR), jnp.int32)]
      + [pltpu.VMEM((_QROWS, _FR), jnp.int32)]
      + [pltpu.VMEM((zcap_rows, _FR), jnp.int32)]
      + [pltpu.VMEM((_FR, _EMBED_DIM), jnp.float32) for _ in range(_NSLOT)]
      + [pltpu.VMEM((_FR, _EMBED_DIM), jnp.float32)]
      + [pltpu.SemaphoreType.DMA for _ in range(2 * _NSLOT + 2)]
      + [pltpu.VMEM_SHARED((_BS, _EMBED_DIM), jnp.float32)]
  )

  @functools.partial(
      pl.kernel,
      out_type=jax.ShapeDtypeStruct((B, _EMBED_DIM), jnp.float32),
      mesh=mesh,
      scratch_types=scratch,
      compiler_params=pltpu.CompilerParams(
          needs_layout_passes=False, use_tc_tiling_on_sc=False),
  )
  def emb(x_hbm, w_hbm, out_hbm, idx_v, gbuf, pbuf, zpbuf, *bufs):
    rows = bufs[:_NSLOT]
    zrows = bufs[_NSLOT]
    gsem = bufs[_NSLOT + 1:2 * _NSLOT + 1]
    wsem = bufs[2 * _NSLOT + 1:3 * _NSLOT + 1]
    zsem = bufs[3 * _NSLOT + 1]
    ssem = bufs[3 * _NSLOT + 2]
    w_sp = bufs[3 * _NSLOT + 3]

    sid = lax.axis_index("s")
    cid = lax.axis_index("c")
    wid = sid * 2 + cid
    base = wid * bpw
    iota = lax.iota(jnp.int32, _L)
    pltpu.sync_copy(x_hbm.at[pl.ds(base, bpw)], idx_v)

    zvec = jnp.zeros((_L,), jnp.float32)

    def zfill(i, c):
      r = jnp.full((_L,), i >> 2, jnp.int32)
      cc = (i & 3) << 4
      plsc.store_scatter(zrows, [r, cc + iota], zvec)
      return c

    lax.fori_loop(0, _FR * _EMBED_DIM // _L, zfill, 0)

    def compact(buf2, vals_fn, mask_fn, glo, ghi, wlo, cap, pos_to=None):
      def grp(i, nv):
        v = idx_v[pl.ds(i * _L, _L)]
        m = mask_fn(v)
        mi = m.astype(jnp.int32)
        cs = plsc.cumsum(mi)
        dst = nv + cs - 1
        sm = m & (dst >= wlo) & (dst < wlo + cap)
        d2 = dst - wlo
        plsc.store_scatter(buf2, [d2 >> 6, d2 & 63], vals_fn(v, i), mask=sm)
        if pos_to is not None:
          posv = base + i * _L + iota
          plsc.store_scatter(pos_to, [d2 >> 6, d2 & 63], posv, mask=sm)
        return nv + jnp.sum(mi)

      return lax.fori_loop(glo, ghi, grp, jnp.int32(0))

    def tail_fill(nv, bufs2):
      last = jnp.maximum(nv - 1, 0)
      lr = jnp.full((_L,), last >> 6, jnp.int32)
      lc = jnp.full((_L,), last & 63, jnp.int32)
      r64 = ((nv + 63) >> 6) << 6
      w0 = nv - (nv & 15)
      for buf2 in bufs2:
        dup = plsc.load_gather(buf2, [lr, lc])
        for t in range(4):
          slot = w0 + t * _L + iota
          mk = (slot >= nv) & (slot < r64)
          plsc.store_scatter(buf2, [slot >> 6, slot & 63], dup, mask=mk)
      return r64 >> 6

    def fire_rounds(nf):
      def fire4(i, c):
        for j in range(_NSLOT):
          k = i * _NSLOT + j

          @pl.when(k < nf)
          def _fire():
            @pl.when(k >= _NSLOT)
            def _wait_prev():
              pltpu.make_async_copy(
                  rows[j], out_hbm.at[pbuf.at[0]], wsem[j]).wait()

            pltpu.async_copy(w_sp.at[gbuf.at[k]], rows[j], gsem[j])
            pltpu.make_async_copy(
                w_sp.at[gbuf.at[k]], rows[j], gsem[j]).wait()
            pltpu.async_copy(rows[j], out_hbm.at[pbuf.at[k]], wsem[j])

        return c

      lax.fori_loop(0, (nf + _NSLOT - 1) // _NSLOT, fire4, 0)

      for j in range(_NSLOT):
        @pl.when(nf > j)
        def _drain():
          pltpu.make_async_copy(
              rows[j], out_hbm.at[pbuf.at[0]], wsem[j]).wait()

    def zwait_one(k, c):
      pltpu.make_async_copy(zrows, out_hbm.at[zpbuf.at[0]], zsem).wait()
      return c

    zpend = jnp.int32(0)
    for p in range(_NBLK):
      lo = _VOCAB_START + p * _BS
      plsc.subcore_barrier()

      @pl.when(sid < _NSTG)
      def _stage():
        pltpu.async_copy(
            w_hbm.at[pl.ds(p * _BS + sid * _SROWS, _SROWS)],
            w_sp.at[pl.ds(sid * _SROWS, _SROWS)], ssem).wait()

      plsc.subcore_barrier()

      in_blk = lambda v: (v >= lo) & (v < lo + _BS)
      to_local = lambda v, i: v - lo
      nv_tot = compact(gbuf, to_local, in_blk, 0, ngrp, jnp.int32(0), q,
                       pos_to=pbuf)
      nv0 = jnp.minimum(nv_tot, q)
      fire_rounds(tail_fill(nv0, [gbuf, pbuf]))

      def extra_round(r, c):
        wlo = r * q
        compact(gbuf, to_local, in_blk, 0, ngrp, wlo, q, pos_to=pbuf)
        nv_r = jnp.minimum(nv_tot - wlo, q)
        fire_rounds(tail_fill(nv_r, [gbuf, pbuf]))
        return c

      lax.fori_loop(1, (nv_tot + q - 1) // q, extra_round, 0)

      lax.fori_loop(0, zpend, zwait_one, 0)
      nz = compact(
          zpbuf,
          lambda v, i: base + i * _L + iota,
          lambda v: (v < _VOCAB_START) | (v >= _VOCAB_END),
          p * zgrp, (p + 1) * zgrp, jnp.int32(0), zgrp * _L)
      nzf = tail_fill(nz, [zpbuf])

      def zfire(k, c):
        pltpu.async_copy(zrows, out_hbm.at[zpbuf.at[k]], zsem)

        @pl.when(k >= _ZWIN)
        def _zw():
          zwait_one(k, 0)

        return c

      lax.fori_loop(0, nzf, zfire, 0)
      zpend = jnp.minimum(nzf, _ZWIN)

    lax.fori_loop(0, zpend, zwait_one, 0)

  return emb


def kernel(x, weight):
  s0, s1 = x.shape
  B = s0 * s1
  xf = x.reshape(B).astype(jnp.int32)
  out = _emb_call(B)(xf, weight)
  return out.reshape(s0, s1, _EMBED_DIM)


# speedup vs baseline: 1.0336x; 1.0161x over previous
import functools

import jax
import jax.numpy as jnp
from jax import lax
from jax.experimental import pallas as pl
from jax.experimental.pallas import tpu as pltpu
from jax.experimental.pallas import tpu_sc as plsc

_NUM_EMBEDDINGS = 1000000
_TP_SIZE = 8
_TP_RANK = 1
_PER_PART = _NUM_EMBEDDINGS // _TP_SIZE
_VOCAB_START = _PER_PART * _TP_RANK
_VOCAB_END = _VOCAB_START + _PER_PART
_EMBED_DIM = 64

_NW = 32
_L = 16
_BS = 15625
_NBLK = _PER_PART // _BS
_NSTG = 5
_SROWS = _BS // _NSTG
_FR = 64
_NSLOT = 4
_ZWIN = 44
_QROWS = 100


def _emb_call(B):
  bpw = B // _NW
  ngrp = bpw // _L
  zgrp = ngrp // _NBLK
  zcap_rows = zgrp * _L // _FR
  q = _QROWS * _FR
  mesh = plsc.VectorSubcoreMesh(core_axis_name="c", subcore_axis_name="s")

  scratch = (
      [pltpu.VMEM((bpw,), jnp.int32)]
      + [pltpu.VMEM((_QROWS, _FR), jnp.int32)]
      + [pltpu.VMEM((_QROWS, _FR), jnp.int32)]
      + [pltpu.VMEM((zcap_rows, _FR), jnp.int32)]
      + [pltpu.VMEM((_FR, _EMBED_DIM), jnp.float32) for _ in range(_NSLOT)]
      + [pltpu.VMEM((_FR, _EMBED_DIM), jnp.float32)]
      + [pltpu.SemaphoreType.DMA for _ in range(2 * _NSLOT + 2)]
      + [pltpu.VMEM_SHARED((_BS, _EMBED_DIM), jnp.float32)]
  )

  @functools.partial(
      pl.kernel,
      out_type=jax.ShapeDtypeStruct((B, _EMBED_DIM), jnp.float32),
      mesh=mesh,
      scratch_types=scratch,
      compiler_params=pltpu.CompilerParams(
          needs_layout_passes=False, use_tc_tiling_on_sc=False),
  )
  def emb(x_hbm, w_hbm, out_hbm, idx_v, gbuf, pbuf, zpbuf, *bufs):
    rows = bufs[:_NSLOT]
    zrows = bufs[_NSLOT]
    gsem = bufs[_NSLOT + 1:2 * _NSLOT + 1]
    wsem = bufs[2 * _NSLOT + 1:3 * _NSLOT + 1]
    zsem = bufs[3 * _NSLOT + 1]
    ssem = bufs[3 * _NSLOT + 2]
    w_sp = bufs[3 * _NSLOT + 3]

    sid = lax.axis_index("s")
    cid = lax.axis_index("c")
    wid = sid * 2 + cid
    base = wid * bpw
    iota = lax.iota(jnp.int32, _L)
    pltpu.sync_copy(x_hbm.at[pl.ds(base, bpw)], idx_v)

    zvec = jnp.zeros((_L,), jnp.float32)

    def zfill(i, c):
      r = jnp.full((_L,), i >> 2, jnp.int32)
      cc = (i & 3) << 4
      plsc.store_scatter(zrows, [r, cc + iota], zvec)
      return c

    lax.fori_loop(0, _FR * _EMBED_DIM // _L, zfill, 0)

    def compact(buf2, vals_fn, mask_fn, glo, ghi, wlo, cap, pos_to=None):
      def grp(i, nv):
        v = idx_v[pl.ds(i * _L, _L)]
        m = mask_fn(v)
        mi = m.astype(jnp.int32)
        cs = plsc.cumsum(mi)
        dst = nv + cs - 1
        sm = m & (dst >= wlo) & (dst < wlo + cap)
        d2 = dst - wlo
        plsc.store_scatter(buf2, [d2 >> 6, d2 & 63], vals_fn(v, i), mask=sm)
        if pos_to is not None:
          posv = base + i * _L + iota
          plsc.store_scatter(pos_to, [d2 >> 6, d2 & 63], posv, mask=sm)
        return nv + jnp.sum(mi)

      return lax.fori_loop(glo, ghi, grp, jnp.int32(0))

    def tail_fill(nv, bufs2):
      last = jnp.maximum(nv - 1, 0)
      lr = jnp.full((_L,), last >> 6, jnp.int32)
      lc = jnp.full((_L,), last & 63, jnp.int32)
      r64 = ((nv + 63) >> 6) << 6
      w0 = nv - (nv & 15)
      for buf2 in bufs2:
        dup = plsc.load_gather(buf2, [lr, lc])
        for t in range(4):
          slot = w0 + t * _L + iota
          mk = (slot >= nv) & (slot < r64)
          plsc.store_scatter(buf2, [slot >> 6, slot & 63], dup, mask=mk)
      return r64 >> 6

    def fire_rounds(nf):
      def fire4(i, c):
        for j in range(_NSLOT):
          k = i * _NSLOT + j

          @pl.when(k < nf)
          def _fire():
            @pl.when(k >= _NSLOT)
            def _wait_prev():
              pltpu.make_async_copy(
                  rows[j], out_hbm.at[pbuf.at[0]], wsem[j]).wait()

            pltpu.async_copy(w_sp.at[gbuf.at[k]], rows[j], gsem[j])
            pltpu.make_async_copy(
                w_sp.at[gbuf.at[k]], rows[j], gsem[j]).wait()
            pltpu.async_copy(rows[j], out_hbm.at[pbuf.at[k]], wsem[j])

        return c

      lax.fori_loop(0, (nf + _NSLOT - 1) // _NSLOT, fire4, 0)

      for j in range(_NSLOT):
        @pl.when(nf > j)
        def _drain():
          pltpu.make_async_copy(
              rows[j], out_hbm.at[pbuf.at[0]], wsem[j]).wait()

    def zwait_one(k, c):
      pltpu.make_async_copy(zrows, out_hbm.at[zpbuf.at[0]], zsem).wait()
      return c

    zpend = jnp.int32(0)
    for p in range(_NBLK):
      lo = _VOCAB_START + p * _BS
      plsc.subcore_barrier()

      @pl.when(sid < _NSTG)
      def _stage():
        pltpu.async_copy(
            w_hbm.at[pl.ds(p * _BS + sid * _SROWS, _SROWS)],
            w_sp.at[pl.ds(sid * _SROWS, _SROWS)], ssem).wait()

      plsc.subcore_barrier()

      in_blk = lambda v: (v >= lo) & (v < lo + _BS)
      to_local = lambda v, i: v - lo
      nv_tot = compact(gbuf, to_local, in_blk, 0, ngrp, jnp.int32(0), q,
                       pos_to=pbuf)
      nv0 = jnp.minimum(nv_tot, q)
      fire_rounds(tail_fill(nv0, [gbuf, pbuf]))

      def extra_round(r, c):
        wlo = r * q
        compact(gbuf, to_local, in_blk, 0, ngrp, wlo, q, pos_to=pbuf)
        nv_r = jnp.minimum(nv_tot - wlo, q)
        fire_rounds(tail_fill(nv_r, [gbuf, pbuf]))
        return c

      lax.fori_loop(1, (nv_tot + q - 1) // q, extra_round, 0)

      lax.fori_loop(0, zpend, zwait_one, 0)
      nz = compact(
          zpbuf,
          lambda v, i: base + i * _L + iota,
          lambda v: (v < _VOCAB_START) | (v >= _VOCAB_END),
          p * zgrp, (p + 1) * zgrp, jnp.int32(0), zgrp * _L)
      nzf = tail_fill(nz, [zpbuf])

      def zfire(k, c):
        pltpu.async_copy(zrows, out_hbm.at[zpbuf.at[k]], zsem)

        @pl.when(k >= _ZWIN)
        def _zw():
          zwait_one(k, 0)

        return c

      lax.fori_loop(0, nzf, zfire, 0)
      zpend = jnp.minimum(nzf, _ZWIN)

    lax.fori_loop(0, zpend, zwait_one, 0)

  return emb


def kernel(x, weight):
  s0, s1 = x.shape
  B = s0 * s1
  xf = x.reshape(B).astype(jnp.int32)
  out = _emb_call(B)(xf, weight)
  return out.reshape(s0, s1, _EMBED_DIM)
